# baseline (speedup 1.0000x reference)
import functools

import jax
import jax.numpy as jnp
from jax import lax
from jax.experimental import pallas as pl
from jax.experimental.pallas import tpu as pltpu
from jax.experimental.pallas import tpu_sc as plsc

B = 16384
D = 32
V = 1000000
RPT = 128 // D
PACKR = V // RPT
FULLC = V // 128
TAILW = V - FULLC * 128
TAILBASE = FULLC * 128
WD = 1e-05
NC = 2
NS = 16
NW = NC * NS
BPW = B // NW
CHUNK = 128
NCHUNK = BPW // CHUNK
SCOLS = 4
SCW = SCOLS * 128
NSUPER = FULLC // SCOLS
KTRIP = 63

_params = pltpu.CompilerParams(
    needs_layout_passes=False, use_tc_tiling_on_sc=True)


def _tp_body(wt_hbm, ht_hbm, wr_hbm, hr_hbm, vin, vout,
             si0, si1, so0, so1):
    cid = lax.axis_index("c")
    sid = lax.axis_index("s")
    wid = sid * NC + cid
    lane = lax.iota(jnp.int32, 16)
    sin = (si0, si1)
    sout = (so0, so1)

    rows_m = [(lane + 16 * m) >> 2 for m in range(8)]
    cols_d = [((lane & 3) * D + d) for d in range(D)]

    def transpose_col(b):
        @plsc.parallel_loop(0, SCW // 16, step=1, unroll=4, carry=jnp.int32(0))
        def _(m, cv):
            rm = (lane + 16 * m) >> 2
            for d in range(D):
                v = vin[b, d, pl.ds(m * 16, 16)]
                plsc.store_scatter(vout.at[b], [rm, cols_d[d]], v)
            return cv

    def do_table(src, dst):
        def cval(k):
            return jnp.minimum(wid + NW * k, NSUPER - 1)

        def in_ref(k):
            off = pl.multiple_of(cval(k) * SCW, 128)
            return src.at[:, pl.ds(off, SCW)]

        def out_ref(k):
            off = pl.multiple_of(cval(k) * SCOLS * D, D)
            return dst.at[pl.ds(off, SCOLS * D), :]

        def issue_in(k, b):
            pltpu.async_copy(in_ref(k), vin.at[b], sin[b])

        def wait_in(k, b):
            pltpu.make_async_copy(in_ref(k), vin.at[b], sin[b]).wait()

        def issue_out(k, b):
            pltpu.async_copy(vout.at[b], out_ref(k), sout[b])

        def wait_out(b):
            pltpu.make_async_copy(vout.at[b], out_ref(0), sout[b]).wait()

        issue_in(0, 0)

        def body(q, carry):
            k0 = 2 * q
            issue_in(k0 + 1, 1)
            wait_in(k0, 0)

            @pl.when(q >= 1)
            def _():
                wait_out(0)

            transpose_col(0)
            issue_out(k0, 0)

            issue_in(k0 + 2, 0)
            wait_in(k0 + 1, 1)

            @pl.when(q >= 1)
            def _():
                wait_out(1)

            transpose_col(1)
            issue_out(k0 + 1, 1)
            return carry

        lax.fori_loop(0, (KTRIP - 1) // 2, body, 0)
        k_last = KTRIP - 1
        wait_in(k_last, 0)
        wait_out(0)
        transpose_col(0)
        issue_out(k_last, 0)
        wait_out(0)
        wait_out(1)

    do_table(wt_hbm, wr_hbm)
    do_table(ht_hbm, hr_hbm)


_tp_call = functools.partial(
    pl.kernel,
    out_type=(jax.ShapeDtypeStruct((PACKR, 128), jnp.float32),
              jax.ShapeDtypeStruct((PACKR, 128), jnp.float32)),
    mesh=plsc.VectorSubcoreMesh(core_axis_name="c", subcore_axis_name="s"),
    compiler_params=_params,
    scratch_types=[
        pltpu.VMEM((2, D, SCW), jnp.float32),
        pltpu.VMEM((2, SCOLS * D, 128), jnp.float32),
        pltpu.SemaphoreType.DMA,
        pltpu.SemaphoreType.DMA,
        pltpu.SemaphoreType.DMA,
        pltpu.SemaphoreType.DMA,
    ],
)(_tp_body)


def _sc_body(u_hbm, i_hbm, j_hbm, w_hbm, h_hbm, wt_tail, ht_tail, out_hbm,
             idx_u, idx_i, idx_j, q4_u, q4_i, q4_j,
             rows_u, rows_i, rows_j, tail_w, tail_h,
             x_v, uu_v, ii_v, jj_v, sem_idx, sem_rows):
    cid = lax.axis_index("c")
    sid = lax.axis_index("s")
    wid = sid * NC + cid
    base = wid * BPW

    pltpu.sync_copy(wt_tail, tail_w)
    pltpu.sync_copy(ht_tail, tail_h)

    idx_copies = []
    for k in range(NCHUNK):
        sl = pl.ds(base + k * CHUNK, CHUNK)
        idx_copies.append(pltpu.async_copy(u_hbm.at[sl], idx_u.at[k], sem_idx))
        idx_copies.append(pltpu.async_copy(i_hbm.at[sl], idx_i.at[k], sem_idx))
        idx_copies.append(pltpu.async_copy(j_hbm.at[sl], idx_j.at[k], sem_idx))
    for c in idx_copies:
        c.wait()

    def prow(v):
        r = lax.shift_left(lax.shift_right_logical(v, 9), 7) | (v & 127)
        return jnp.minimum(r, NSUPER * SCOLS * D - 1)

    for k in range(NCHUNK):
        for o in range(0, CHUNK, 16):
            sl = pl.ds(o, 16)
            q4_u[k, sl] = prow(idx_u[k, sl])
            q4_i[k, sl] = prow(idx_i[k, sl])
            q4_j[k, sl] = prow(idx_j[k, sl])

    lane = lax.iota(jnp.int32, 16)

    def start(c):
        b = c % 2
        return [
            pltpu.async_copy(w_hbm.at[q4_u.at[c]], rows_u.at[b], sem_rows),
            pltpu.async_copy(h_hbm.at[q4_i.at[c]], rows_i.at[b], sem_rows),
            pltpu.async_copy(h_hbm.at[q4_j.at[c]], rows_j.at[b], sem_rows),
        ]

    pending = start(0)
    for c in range(NCHUNK):
        nxt = start(c + 1) if c + 1 < NCHUNK else []
        for cp in pending:
            cp.wait()
        pending = nxt
        b = c % 2
        ru, ri, rj = rows_u.at[b], rows_i.at[b], rows_j.at[b]

        def group(g, carry):
            row_ids = g * 16 + lane
            sl16 = pl.ds(g * 16, 16)
            iu = idx_u[c, sl16]
            ii_ = idx_i[c, sl16]
            ij = idx_j[c, sl16]
            qu = (lax.shift_right_logical(iu, 7) & 3) * D
            qi = (lax.shift_right_logical(ii_, 7) & 3) * D
            qj = (lax.shift_right_logical(ij, 7) & 3) * D
            mu = iu >= TAILBASE
            mi = ii_ >= TAILBASE
            mj = ij >= TAILBASE
            tu = jnp.minimum(jnp.maximum(iu - TAILBASE, 0), TAILW - 1)
            ti = jnp.minimum(jnp.maximum(ii_ - TAILBASE, 0), TAILW - 1)
            tj = jnp.minimum(jnp.maximum(ij - TAILBASE, 0), TAILW - 1)
            xa = jnp.zeros((16,), jnp.float32)
            ua = jnp.zeros((16,), jnp.float32)
            ia = jnp.zeros((16,), jnp.float32)
            ja = jnp.zeros((16,), jnp.float32)
            for d in range(D):
                dv = jnp.full((16,), d, jnp.int32)
                cu = plsc.load_gather(ru, [row_ids, qu + d])
                ci = plsc.load_gather(ri, [row_ids, qi + d])
                cj = plsc.load_gather(rj, [row_ids, qj + d])
                cu = jnp.where(mu, plsc.load_gather(tail_w, [tu, dv]), cu)
                ci = jnp.where(mi, plsc.load_gather(tail_h, [ti, dv]), ci)
                cj = jnp.where(mj, plsc.load_gather(tail_h, [tj, dv]), cj)
                xa = xa + cu * (ci - cj)
                ua = ua + cu * cu
                ia = ia + ci * ci
                ja = ja + cj * cj
            osl = pl.ds(c * CHUNK + g * 16, 16)
            x_v[osl] = xa
            uu_v[osl] = ua
            ii_v[osl] = ia
            jj_v[osl] = ja
            return carry

        lax.fori_loop(0, CHUNK // 16, group, 0)

    pltpu.sync_copy(x_v, out_hbm.at[pl.ds(0 * B + base, BPW)])
    pltpu.sync_copy(uu_v, out_hbm.at[pl.ds(1 * B + base, BPW)])
    pltpu.sync_copy(ii_v, out_hbm.at[pl.ds(2 * B + base, BPW)])
    pltpu.sync_copy(jj_v, out_hbm.at[pl.ds(3 * B + base, BPW)])


_sc_call = functools.partial(
    pl.kernel,
    out_type=jax.ShapeDtypeStruct((4 * B,), jnp.float32),
    mesh=plsc.VectorSubcoreMesh(core_axis_name="c", subcore_axis_name="s"),
    compiler_params=_params,
    scratch_types=[
        pltpu.VMEM((NCHUNK, CHUNK), jnp.int32),
        pltpu.VMEM((NCHUNK, CHUNK), jnp.int32),
        pltpu.VMEM((NCHUNK, CHUNK), jnp.int32),
        pltpu.VMEM((NCHUNK, CHUNK), jnp.int32),
        pltpu.VMEM((NCHUNK, CHUNK), jnp.int32),
        pltpu.VMEM((NCHUNK, CHUNK), jnp.int32),
        pltpu.VMEM((2, CHUNK, 128), jnp.float32),
        pltpu.VMEM((2, CHUNK, 128), jnp.float32),
        pltpu.VMEM((2, CHUNK, 128), jnp.float32),
        pltpu.VMEM((TAILW, 128), jnp.float32),
        pltpu.VMEM((TAILW, 128), jnp.float32),
        pltpu.VMEM((BPW,), jnp.float32),
        pltpu.VMEM((BPW,), jnp.float32),
        pltpu.VMEM((BPW,), jnp.float32),
        pltpu.VMEM((BPW,), jnp.float32),
        pltpu.SemaphoreType.DMA,
        pltpu.SemaphoreType.DMA,
    ],
)(_sc_body)


def _tcp_body(in_ref, out_ref):
    t = in_ref[...].T
    out_ref[...] = jnp.concatenate(
        [t[q * 128:(q + 1) * 128, :] for q in range(SCOLS)], axis=1)


_tcp_call = pl.pallas_call(
    _tcp_body,
    grid=(NSUPER,),
    in_specs=[pl.BlockSpec((D, SCW), lambda g: (0, g))],
    out_specs=pl.BlockSpec((SCOLS * D, 128), lambda g: (g, 0)),
    out_shape=jax.ShapeDtypeStruct((NSUPER * SCOLS * D, 128), jnp.float32),
)


def _tc_body(o_ref, out_ref):
    x = o_ref[pl.ds(0, 128), :]
    uu = o_ref[pl.ds(128, 128), :]
    ii = o_ref[pl.ds(256, 128), :]
    jj = o_ref[pl.ds(384, 128), :]
    reg = WD * (jnp.sqrt(uu) + jnp.sqrt(ii) + jnp.sqrt(jj))
    out_ref[...] = -jax.nn.log_sigmoid(x) + reg


_tc_call = pl.pallas_call(
    _tc_body,
    out_shape=jax.ShapeDtypeStruct((128, 128), jnp.float32),
)


def kernel(u, i, j, W, H):
    u = u.astype(jnp.int32)
    i = i.astype(jnp.int32)
    j = j.astype(jnp.int32)
    wt_tail = jnp.pad(W[TAILBASE:, :], ((0, 0), (0, 128 - D)))
    ht_tail = jnp.pad(H[TAILBASE:, :], ((0, 0), (0, 128 - D)))
    Wr = _tcp_call(W.T)
    Hr = _tcp_call(H.T)
    packed = _sc_call(u, i, j, Wr, Hr, wt_tail, ht_tail)
    return _tc_call(packed.reshape(512, 128)).reshape(B)


# speedup vs baseline: 2.5759x; 2.5759x over previous
import functools

import jax
import jax.numpy as jnp
from jax import lax
from jax.experimental import pallas as pl
from jax.experimental.pallas import tpu as pltpu
from jax.experimental.pallas import tpu_sc as plsc

B = 16384
D = 32
RPT = 128 // D
WD = 1e-05
NC = 2
NS = 16
NW = NC * NS
BPW = B // NW
CHUNK = 128
NCHUNK = BPW // CHUNK


def _sc_body(u_hbm, i_hbm, j_hbm, w_hbm, h_hbm, out_hbm,
             idx_u, idx_i, idx_j, q4_u, q4_i, q4_j,
             rows_u, rows_i, rows_j,
             x_v, uu_v, ii_v, jj_v, sem_idx, sem_rows):
    cid = lax.axis_index("c")
    sid = lax.axis_index("s")
    wid = sid * NC + cid
    base = wid * BPW

    idx_copies = []
    for k in range(NCHUNK):
        sl = pl.ds(base + k * CHUNK, CHUNK)
        idx_copies.append(pltpu.async_copy(u_hbm.at[sl], idx_u.at[k], sem_idx))
        idx_copies.append(pltpu.async_copy(i_hbm.at[sl], idx_i.at[k], sem_idx))
        idx_copies.append(pltpu.async_copy(j_hbm.at[sl], idx_j.at[k], sem_idx))
    for c in idx_copies:
        c.wait()

    for k in range(NCHUNK):
        for o in range(0, CHUNK, 16):
            sl = pl.ds(o, 16)
            q4_u[k, sl] = lax.shift_right_logical(idx_u[k, sl], 2)
            q4_i[k, sl] = lax.shift_right_logical(idx_i[k, sl], 2)
            q4_j[k, sl] = lax.shift_right_logical(idx_j[k, sl], 2)

    lane = lax.iota(jnp.int32, 16)

    def start(c):
        b = c % 2
        return [
            pltpu.async_copy(w_hbm.at[q4_u.at[c]], rows_u.at[b], sem_rows),
            pltpu.async_copy(h_hbm.at[q4_i.at[c]], rows_i.at[b], sem_rows),
            pltpu.async_copy(h_hbm.at[q4_j.at[c]], rows_j.at[b], sem_rows),
        ]

    pending = start(0)
    for c in range(NCHUNK):
        nxt = start(c + 1) if c + 1 < NCHUNK else []
        for cp in pending:
            cp.wait()
        pending = nxt
        b = c % 2
        ru, ri, rj = rows_u.at[b], rows_i.at[b], rows_j.at[b]

        def group(g, carry):
            row_ids = g * 16 + lane
            sl16 = pl.ds(g * 16, 16)
            qu = (idx_u[c, sl16] & 3) * D
            qi = (idx_i[c, sl16] & 3) * D
            qj = (idx_j[c, sl16] & 3) * D
            xa = jnp.zeros((16,), jnp.float32)
            ua = jnp.zeros((16,), jnp.float32)
            ia = jnp.zeros((16,), jnp.float32)
            ja = jnp.zeros((16,), jnp.float32)
            for d in range(D):
                cu = plsc.load_gather(ru, [row_ids, qu + d])
                ci = plsc.load_gather(ri, [row_ids, qi + d])
                cj = plsc.load_gather(rj, [row_ids, qj + d])
                xa = xa + cu * (ci - cj)
                ua = ua + cu * cu
                ia = ia + ci * ci
                ja = ja + cj * cj
            osl = pl.ds(c * CHUNK + g * 16, 16)
            x_v[osl] = xa
            uu_v[osl] = ua
            ii_v[osl] = ia
            jj_v[osl] = ja
            return carry

        lax.fori_loop(0, CHUNK // 16, group, 0)

    pltpu.sync_copy(x_v, out_hbm.at[pl.ds(0 * B + base, BPW)])
    pltpu.sync_copy(uu_v, out_hbm.at[pl.ds(1 * B + base, BPW)])
    pltpu.sync_copy(ii_v, out_hbm.at[pl.ds(2 * B + base, BPW)])
    pltpu.sync_copy(jj_v, out_hbm.at[pl.ds(3 * B + base, BPW)])


_sc_call = functools.partial(
    pl.kernel,
    out_type=jax.ShapeDtypeStruct((4 * B,), jnp.float32),
    mesh=plsc.VectorSubcoreMesh(core_axis_name="c", subcore_axis_name="s"),
    compiler_params=pltpu.CompilerParams(
        needs_layout_passes=False, use_tc_tiling_on_sc=True),
    scratch_types=[
        pltpu.VMEM((NCHUNK, CHUNK), jnp.int32),
        pltpu.VMEM((NCHUNK, CHUNK), jnp.int32),
        pltpu.VMEM((NCHUNK, CHUNK), jnp.int32),
        pltpu.VMEM((NCHUNK, CHUNK), jnp.int32),
        pltpu.VMEM((NCHUNK, CHUNK), jnp.int32),
        pltpu.VMEM((NCHUNK, CHUNK), jnp.int32),
        pltpu.VMEM((2, CHUNK, 128), jnp.float32),
        pltpu.VMEM((2, CHUNK, 128), jnp.float32),
        pltpu.VMEM((2, CHUNK, 128), jnp.float32),
        pltpu.VMEM((BPW,), jnp.float32),
        pltpu.VMEM((BPW,), jnp.float32),
        pltpu.VMEM((BPW,), jnp.float32),
        pltpu.VMEM((BPW,), jnp.float32),
        pltpu.SemaphoreType.DMA,
        pltpu.SemaphoreType.DMA,
    ],
)(_sc_body)


def _tc_body(o_ref, out_ref):
    x = o_ref[pl.ds(0, 128), :]
    uu = o_ref[pl.ds(128, 128), :]
    ii = o_ref[pl.ds(256, 128), :]
    jj = o_ref[pl.ds(384, 128), :]
    reg = WD * (jnp.sqrt(uu) + jnp.sqrt(ii) + jnp.sqrt(jj))
    out_ref[...] = -jax.nn.log_sigmoid(x) + reg


_tc_call = pl.pallas_call(
    _tc_body,
    out_shape=jax.ShapeDtypeStruct((128, 128), jnp.float32),
)


def kernel(u, i, j, W, H):
    u = u.astype(jnp.int32)
    i = i.astype(jnp.int32)
    j = j.astype(jnp.int32)
    Wr = W.reshape(W.shape[0] // RPT, 128)
    Hr = H.reshape(H.shape[0] // RPT, 128)
    packed = _sc_call(u, i, j, Wr, Hr)
    return _tc_call(packed.reshape(512, 128)).reshape(B)
